# baseline (speedup 1.0000x reference)
import functools

import jax
import jax.numpy as jnp
from jax import lax
from jax.experimental import pallas as pl
from jax.experimental.pallas import tpu as pltpu
from jax.experimental.pallas import tpu_sc as plsc

_D = 128
_SEQ = 50
_SEQ_PAD = 56
_B = 4096
_OUT = 512
_NC = 2
_NS = 16
_NW = _NC * _NS
_ROWS_PER_TILE = _B // _NW
_ROWS_PER_GATHER = 2
_IDX_PER_GATHER = _ROWS_PER_GATHER * _SEQ_PAD
_GATHERS_PER_TILE = _ROWS_PER_TILE // _ROWS_PER_GATHER
_LANES = 16


def _sc_pool(table, idx_flat):
    mesh = plsc.VectorSubcoreMesh(core_axis_name="c", subcore_axis_name="s")

    @functools.partial(
        pl.kernel,
        mesh=mesh,
        out_type=jax.ShapeDtypeStruct((_B, _D), jnp.float32),
        scratch_types=[
            pltpu.VMEM((_ROWS_PER_TILE * _SEQ_PAD,), jnp.int32),
            pltpu.VMEM((_IDX_PER_GATHER, _D), jnp.float32),
            pltpu.VMEM((_ROWS_PER_TILE, _D), jnp.float32),
            pltpu.SemaphoreType.DMA,
        ],
    )
    def k(table_hbm, idx_hbm, out_hbm, idx_v, rows_v, out_v, sem):
        wid = lax.axis_index("s") * _NC + lax.axis_index("c")
        row0 = wid * _ROWS_PER_TILE
        pltpu.sync_copy(
            idx_hbm.at[pl.ds(row0 * _SEQ_PAD, _ROWS_PER_TILE * _SEQ_PAD)],
            idx_v)

        @pl.loop(0, _GATHERS_PER_TILE)
        def _gather_chunk(g):
            pltpu.async_copy(
                table_hbm.at[idx_v.at[pl.ds(g * _IDX_PER_GATHER,
                                            _IDX_PER_GATHER)]],
                rows_v, sem).wait()
            for r in range(_ROWS_PER_GATHER):
                @pl.loop(0, _D // _LANES)
                def _reduce(cc, r=r):
                    c0 = cc * _LANES
                    acc = rows_v[r * _SEQ_PAD, pl.ds(c0, _LANES)]
                    for l in range(1, _SEQ):
                        acc = acc + rows_v[r * _SEQ_PAD + l, pl.ds(c0, _LANES)]
                    out_v[g * _ROWS_PER_GATHER + r, pl.ds(c0, _LANES)] = (
                        acc * (1.0 / _SEQ))

        pltpu.sync_copy(out_v, out_hbm.at[pl.ds(row0, _ROWS_PER_TILE)])

    return k(table, idx_flat)


def _tc_project(pooled, w, bias):
    blk = 512

    def body(p_ref, w_ref, b_ref, o_ref):
        o_ref[...] = lax.dot_general(
            p_ref[...], w_ref[...], (((1,), (0,)), ((), ())),
            preferred_element_type=jnp.float32,
            precision=lax.Precision.HIGHEST) + b_ref[...]

    return pl.pallas_call(
        body,
        grid=(_B // blk,),
        in_specs=[
            pl.BlockSpec((blk, _D), lambda i: (i, 0)),
            pl.BlockSpec((_D, _OUT), lambda i: (0, 0)),
            pl.BlockSpec((1, _OUT), lambda i: (0, 0)),
        ],
        out_specs=pl.BlockSpec((blk, _OUT), lambda i: (i, 0)),
        out_shape=jax.ShapeDtypeStruct((_B, _OUT), jnp.float32),
    )(pooled, w, bias.reshape(1, _OUT))


def kernel(x, table, W, b):
    xp = jnp.pad(x.astype(jnp.int32), ((0, 0), (0, _SEQ_PAD - _SEQ)))
    pooled = _sc_pool(table, xp.reshape(-1))
    return _tc_project(pooled, W, b)


# speedup vs baseline: 1.0006x; 1.0006x over previous
import functools

import jax
import jax.numpy as jnp
from jax import lax
from jax.experimental import pallas as pl
from jax.experimental.pallas import tpu as pltpu
from jax.experimental.pallas import tpu_sc as plsc

_D = 128
_SEQ = 50
_SEQ_PAD = 56
_B = 4096
_OUT = 512
_NC = 2
_NS = 16
_NW = _NC * _NS
_ROWS_PER_TILE = _B // _NW
_ROWS_PER_GATHER = 2
_IDX_PER_GATHER = _ROWS_PER_GATHER * _SEQ_PAD
_GATHERS_PER_TILE = _ROWS_PER_TILE // _ROWS_PER_GATHER
_LANES = 16


_NBUF = 4


def _sc_pool(table, idx_flat):
    mesh = plsc.VectorSubcoreMesh(core_axis_name="c", subcore_axis_name="s")

    @functools.partial(
        pl.kernel,
        mesh=mesh,
        out_type=jax.ShapeDtypeStruct((_B, _D), jnp.float32),
        scratch_types=(
            [pltpu.VMEM((_ROWS_PER_TILE * _SEQ_PAD,), jnp.int32)]
            + [pltpu.VMEM((_IDX_PER_GATHER, _D), jnp.float32)] * _NBUF
            + [pltpu.VMEM((_ROWS_PER_TILE, _D), jnp.float32)]
            + [pltpu.SemaphoreType.DMA] * _NBUF
        ),
    )
    def k(table_hbm, idx_hbm, out_hbm, idx_v, *rest):
        bufs = rest[:_NBUF]
        out_v = rest[_NBUF]
        sems = rest[_NBUF + 1:]
        wid = lax.axis_index("s") * _NC + lax.axis_index("c")
        row0 = wid * _ROWS_PER_TILE
        pltpu.sync_copy(
            idx_hbm.at[pl.ds(row0 * _SEQ_PAD, _ROWS_PER_TILE * _SEQ_PAD)],
            idx_v)

        def src(g):
            return table_hbm.at[idx_v.at[pl.ds(g * _IDX_PER_GATHER,
                                               _IDX_PER_GATHER)]]

        def fire(g, b):
            pltpu.async_copy(src(g), bufs[b], sems[b])

        def wait(g, b):
            pltpu.make_async_copy(src(g), bufs[b], sems[b]).wait()

        def reduce(buf, g):
            @pl.loop(0, _D // _LANES)
            def _reduce(cc):
                c0 = cc * _LANES
                for r in range(_ROWS_PER_GATHER):
                    base = r * _SEQ_PAD
                    accs = [buf[base + a, pl.ds(c0, _LANES)] for a in range(4)]
                    for l in range(4, _SEQ):
                        accs[l % 4] = accs[l % 4] + buf[base + l,
                                                        pl.ds(c0, _LANES)]
                    s = (accs[0] + accs[1]) + (accs[2] + accs[3])
                    out_v[g * _ROWS_PER_GATHER + r, pl.ds(c0, _LANES)] = (
                        s * (1.0 / _SEQ))

        for b in range(_NBUF):
            fire(b, b)

        @pl.loop(0, _GATHERS_PER_TILE - _NBUF, step=_NBUF)
        def _main(g0):
            for b in range(_NBUF):
                g = g0 + b
                wait(g, b)
                reduce(bufs[b], g)
                fire(g + _NBUF, b)

        for b in range(_NBUF):
            g = _GATHERS_PER_TILE - _NBUF + b
            wait(g, b)
            reduce(bufs[b], g)

        pltpu.sync_copy(out_v, out_hbm.at[pl.ds(row0, _ROWS_PER_TILE)])

    return k(table, idx_flat)


def _tc_project(pooled, w, bias):
    blk = 512

    def body(p_ref, w_ref, b_ref, o_ref):
        o_ref[...] = lax.dot_general(
            p_ref[...], w_ref[...], (((1,), (0,)), ((), ())),
            preferred_element_type=jnp.float32,
            precision=lax.Precision.HIGHEST) + b_ref[...]

    return pl.pallas_call(
        body,
        grid=(_B // blk,),
        in_specs=[
            pl.BlockSpec((blk, _D), lambda i: (i, 0)),
            pl.BlockSpec((_D, _OUT), lambda i: (0, 0)),
            pl.BlockSpec((1, _OUT), lambda i: (0, 0)),
        ],
        out_specs=pl.BlockSpec((blk, _OUT), lambda i: (i, 0)),
        out_shape=jax.ShapeDtypeStruct((_B, _OUT), jnp.float32),
    )(pooled, w, bias.reshape(1, _OUT))


def kernel(x, table, W, b):
    xp = jnp.pad(x.astype(jnp.int32), ((0, 0), (0, _SEQ_PAD - _SEQ)))
    pooled = _sc_pool(table, xp.reshape(-1))
    return _tc_project(pooled, W, b)


# speedup vs baseline: 1.0148x; 1.0142x over previous
import dataclasses
import functools

import jax
import jax.numpy as jnp
import numpy as np
from jax import lax
from jax.experimental import pallas as pl
from jax.experimental.pallas import tpu as pltpu
from jax.experimental.pallas import tpu_sc as plsc

_D = 128
_SEQ = 50
_SEQ_PAD = 56
_B = 4096
_OUT = 512
_NC = 2
_NS = 16
_NW = _NC * _NS
_ROWS_PER_TILE = _B // _NW
_ROWS_PER_GATHER = 2
_IDX_PER_GATHER = _ROWS_PER_GATHER * _SEQ_PAD
_GATHERS_PER_TILE = _ROWS_PER_TILE // _ROWS_PER_GATHER
_LANES = 16
_NBUF = 4

_PERM = np.concatenate(
    [np.concatenate([g * 32 + np.arange(0, 32, 2),
                     g * 32 + np.arange(1, 32, 2)])
     for g in range(_D // 32)])


def _sc_pool(table_bf16, idx_flat):
    mesh = plsc.VectorSubcoreMesh(core_axis_name="c", subcore_axis_name="s")
    cp = pltpu.CompilerParams()
    if "needs_layout_passes" in pltpu.CompilerParams.__dataclass_fields__:
        cp = dataclasses.replace(cp, needs_layout_passes=False)
    if "use_tc_tiling_on_sc" in pltpu.CompilerParams.__dataclass_fields__:
        cp = dataclasses.replace(cp, use_tc_tiling_on_sc=False)

    @functools.partial(
        pl.kernel,
        mesh=mesh,
        compiler_params=cp,
        out_type=jax.ShapeDtypeStruct((_B, _D), jnp.float32),
        scratch_types=(
            [pltpu.VMEM((_ROWS_PER_TILE * _SEQ_PAD,), jnp.int32)]
            + [pltpu.VMEM((_IDX_PER_GATHER, _D // 2), jnp.float32)] * _NBUF
            + [pltpu.VMEM((_ROWS_PER_TILE, _D), jnp.float32)]
            + [pltpu.SemaphoreType.DMA] * _NBUF
        ),
    )
    def k(table_hbm, idx_hbm, out_hbm, idx_v, *rest):
        bufs = rest[:_NBUF]
        out_v = rest[_NBUF]
        sems = rest[_NBUF + 1:]
        wid = lax.axis_index("s") * _NC + lax.axis_index("c")
        row0 = wid * _ROWS_PER_TILE
        pltpu.sync_copy(
            idx_hbm.at[pl.ds(row0 * _SEQ_PAD, _ROWS_PER_TILE * _SEQ_PAD)],
            idx_v)

        def src(g):
            return table_hbm.at[idx_v.at[pl.ds(g * _IDX_PER_GATHER,
                                               _IDX_PER_GATHER)]]

        def fire(g, b):
            pltpu.async_copy(src(g), bufs[b], sems[b])

        def wait(g, b):
            pltpu.make_async_copy(src(g), bufs[b], sems[b]).wait()

        def reduce(buf, g):
            @pl.loop(0, _D // 32)
            def _reduce(gg):
                c0 = gg * 32
                for r in range(_ROWS_PER_GATHER):
                    base = r * _SEQ_PAD
                    ae = [None, None]
                    ao = [None, None]
                    for l in range(_SEQ):
                        v = plsc.bitcast(
                            buf[base + l, pl.ds(c0 // 2, _LANES)],
                            jnp.bfloat16)
                        e, o = plsc.unpack(
                            v, format=plsc.PackFormat.INTERLEAVED,
                            preferred_element_type=jnp.float32)
                        i = l % 2
                        ae[i] = e if ae[i] is None else ae[i] + e
                        ao[i] = o if ao[i] is None else ao[i] + o
                    row_out = g * _ROWS_PER_GATHER + r
                    out_v[row_out, pl.ds(c0, _LANES)] = (
                        (ae[0] + ae[1]) * (1.0 / _SEQ))
                    out_v[row_out, pl.ds(c0 + _LANES, _LANES)] = (
                        (ao[0] + ao[1]) * (1.0 / _SEQ))

        for b in range(_NBUF):
            fire(b, b)

        @pl.loop(0, _GATHERS_PER_TILE - _NBUF, step=_NBUF)
        def _main(g0):
            for b in range(_NBUF):
                g = g0 + b
                wait(g, b)
                reduce(bufs[b], g)
                fire(g + _NBUF, b)

        for b in range(_NBUF):
            g = _GATHERS_PER_TILE - _NBUF + b
            wait(g, b)
            reduce(bufs[b], g)

        pltpu.sync_copy(out_v, out_hbm.at[pl.ds(row0, _ROWS_PER_TILE)])

    return k(table_bf16, idx_flat)


def _tc_project(pooled, w, bias):
    blk = 512

    def body(p_ref, w_ref, b_ref, o_ref):
        o_ref[...] = lax.dot_general(
            p_ref[...], w_ref[...], (((1,), (0,)), ((), ())),
            preferred_element_type=jnp.float32,
            precision=lax.Precision.HIGHEST) + b_ref[...]

    return pl.pallas_call(
        body,
        grid=(_B // blk,),
        in_specs=[
            pl.BlockSpec((blk, _D), lambda i: (i, 0)),
            pl.BlockSpec((_D, _OUT), lambda i: (0, 0)),
            pl.BlockSpec((1, _OUT), lambda i: (0, 0)),
        ],
        out_specs=pl.BlockSpec((blk, _OUT), lambda i: (i, 0)),
        out_shape=jax.ShapeDtypeStruct((_B, _OUT), jnp.float32),
    )(pooled, w, bias.reshape(1, _OUT))


def kernel(x, table, W, b):
    xp = jnp.pad(x.astype(jnp.int32), ((0, 0), (0, _SEQ_PAD - _SEQ)))
    table_w = lax.bitcast_convert_type(
        table.astype(jnp.bfloat16).reshape(-1, _D // 2, 2), jnp.float32)
    pooled_perm = _sc_pool(table_w, xp.reshape(-1))
    return _tc_project(pooled_perm, W[_PERM], b)


# speedup vs baseline: 1.0174x; 1.0026x over previous
import dataclasses
import functools

import jax
import jax.numpy as jnp
import numpy as np
from jax import lax
from jax.experimental import pallas as pl
from jax.experimental.pallas import tpu as pltpu
from jax.experimental.pallas import tpu_sc as plsc

_D = 128
_SEQ = 50
_SEQ_PAD = 56
_B = 4096
_OUT = 512
_NC = 2
_NS = 16
_NW = _NC * _NS
_ROWS_PER_TILE = _B // _NW
_ROWS_PER_GATHER = 2
_IDX_PER_GATHER = _ROWS_PER_GATHER * _SEQ_PAD
_GATHERS_PER_TILE = _ROWS_PER_TILE // _ROWS_PER_GATHER
_LANES = 16
_NBUF = 4

_PERM = np.concatenate(
    [np.concatenate([g * 32 + np.arange(0, 32, 2),
                     g * 32 + np.arange(1, 32, 2)])
     for g in range(_D // 32)])


def _sc_pool(table_bf16, idx_flat):
    mesh = plsc.VectorSubcoreMesh(core_axis_name="c", subcore_axis_name="s")
    cp = pltpu.CompilerParams()
    if "needs_layout_passes" in pltpu.CompilerParams.__dataclass_fields__:
        cp = dataclasses.replace(cp, needs_layout_passes=False)
    if "use_tc_tiling_on_sc" in pltpu.CompilerParams.__dataclass_fields__:
        cp = dataclasses.replace(cp, use_tc_tiling_on_sc=False)

    @functools.partial(
        pl.kernel,
        mesh=mesh,
        compiler_params=cp,
        out_type=jax.ShapeDtypeStruct((_B, _D), jnp.float32),
        scratch_types=(
            [pltpu.VMEM((_GATHERS_PER_TILE, _IDX_PER_GATHER), jnp.int32)]
            + [pltpu.VMEM((_IDX_PER_GATHER, _D // 2), jnp.float32)] * _NBUF
            + [pltpu.VMEM((_ROWS_PER_TILE, _D), jnp.float32)]
            + [pltpu.SemaphoreType.DMA] * _NBUF
        ),
    )
    def k(table_hbm, idx_hbm, out_hbm, idx_v, *rest):
        bufs = rest[:_NBUF]
        out_v = rest[_NBUF]
        sems = rest[_NBUF + 1:]
        wid = lax.axis_index("s") * _NC + lax.axis_index("c")
        row0 = wid * _ROWS_PER_TILE
        pltpu.sync_copy(
            idx_hbm.at[pl.ds(wid * _GATHERS_PER_TILE, _GATHERS_PER_TILE)],
            idx_v)

        def src(g):
            return table_hbm.at[idx_v.at[g]]

        def fire(g, b):
            pltpu.async_copy(src(g), bufs[b], sems[b])

        def wait(g, b):
            pltpu.make_async_copy(src(g), bufs[b], sems[b]).wait()

        def reduce(buf, g):
            @pl.loop(0, _D // 32)
            def _reduce(gg):
                c0 = gg * 32
                for r in range(_ROWS_PER_GATHER):
                    base = r * _SEQ_PAD
                    ae = [None, None]
                    ao = [None, None]
                    for l in range(_SEQ):
                        v = plsc.bitcast(
                            buf[base + l, pl.ds(c0 // 2, _LANES)],
                            jnp.bfloat16)
                        e, o = plsc.unpack(
                            v, format=plsc.PackFormat.INTERLEAVED,
                            preferred_element_type=jnp.float32)
                        i = l % 2
                        ae[i] = e if ae[i] is None else ae[i] + e
                        ao[i] = o if ao[i] is None else ao[i] + o
                    row_out = g * _ROWS_PER_GATHER + r
                    out_v[row_out, pl.ds(c0, _LANES)] = (
                        (ae[0] + ae[1]) * (1.0 / _SEQ))
                    out_v[row_out, pl.ds(c0 + _LANES, _LANES)] = (
                        (ao[0] + ao[1]) * (1.0 / _SEQ))

        for b in range(_NBUF):
            fire(b, b)

        @pl.loop(0, _GATHERS_PER_TILE - _NBUF, step=_NBUF)
        def _main(g0):
            for b in range(_NBUF):
                g = g0 + b
                wait(g, b)
                reduce(bufs[b], g)
                fire(g + _NBUF, b)

        for b in range(_NBUF):
            g = _GATHERS_PER_TILE - _NBUF + b
            wait(g, b)
            reduce(bufs[b], g)

        pltpu.sync_copy(out_v, out_hbm.at[pl.ds(row0, _ROWS_PER_TILE)])

    return k(table_bf16, idx_flat)


def _tc_project(pooled, w, bias):
    blk = 512

    def body(p_ref, w_ref, b_ref, o_ref):
        o_ref[...] = lax.dot_general(
            p_ref[...], w_ref[...], (((1,), (0,)), ((), ())),
            preferred_element_type=jnp.float32,
            precision=lax.Precision.HIGHEST) + b_ref[...]

    return pl.pallas_call(
        body,
        grid=(_B // blk,),
        in_specs=[
            pl.BlockSpec((blk, _D), lambda i: (i, 0)),
            pl.BlockSpec((_D, _OUT), lambda i: (0, 0)),
            pl.BlockSpec((1, _OUT), lambda i: (0, 0)),
        ],
        out_specs=pl.BlockSpec((blk, _OUT), lambda i: (i, 0)),
        out_shape=jax.ShapeDtypeStruct((_B, _OUT), jnp.float32),
    )(pooled, w, bias.reshape(1, _OUT))


def kernel(x, table, W, b):
    xp = jnp.pad(x.astype(jnp.int32), ((0, 0), (0, _SEQ_PAD - _SEQ)))
    xp = xp.reshape(_B // _ROWS_PER_GATHER, _IDX_PER_GATHER)
    table_w = lax.bitcast_convert_type(
        table.astype(jnp.bfloat16).reshape(-1, _D // 2, 2), jnp.float32)
    pooled_perm = _sc_pool(table_w, xp)
    return _tc_project(pooled_perm, W[_PERM], b)


# speedup vs baseline: 1.8800x; 1.8478x over previous
import dataclasses
import functools

import jax
import jax.numpy as jnp
import numpy as np
from jax import lax
from jax.experimental import pallas as pl
from jax.experimental.pallas import tpu as pltpu
from jax.experimental.pallas import tpu_sc as plsc

_D = 128
_SEQ = 50
_B = 4096
_OUT = 512
_NC = 2
_NS = 16
_NW = _NC * _NS
_ROWS_PER_TILE = _B // _NW
_ROWS_PER_GATHER = 2
_IDX_PER_GATHER = _ROWS_PER_GATHER * _SEQ
_GATHERS_PER_TILE = _ROWS_PER_TILE // _ROWS_PER_GATHER
_LANES = 16
_NBUF = 4

_PERM = np.concatenate(
    [np.concatenate([g * 32 + np.arange(0, 32, 2),
                     g * 32 + np.arange(1, 32, 2)])
     for g in range(_D // 32)])


def _sc_pool(table_bf16, idx_flat):
    mesh = plsc.VectorSubcoreMesh(core_axis_name="c", subcore_axis_name="s")
    cp = pltpu.CompilerParams()
    if "needs_layout_passes" in pltpu.CompilerParams.__dataclass_fields__:
        cp = dataclasses.replace(cp, needs_layout_passes=False)
    if "use_tc_tiling_on_sc" in pltpu.CompilerParams.__dataclass_fields__:
        cp = dataclasses.replace(cp, use_tc_tiling_on_sc=False)

    @functools.partial(
        pl.kernel,
        mesh=mesh,
        compiler_params=cp,
        out_type=jax.ShapeDtypeStruct((_B, _D), jnp.float32),
        scratch_types=(
            [pltpu.VMEM((_GATHERS_PER_TILE, _IDX_PER_GATHER), jnp.int32)]
            + [pltpu.VMEM((_IDX_PER_GATHER, _D // 2), jnp.float32)] * _NBUF
            + [pltpu.VMEM((_ROWS_PER_TILE, _D), jnp.float32)]
            + [pltpu.SemaphoreType.DMA] * _NBUF
        ),
    )
    def k(table_hbm, idx_hbm, out_hbm, idx_v, *rest):
        bufs = rest[:_NBUF]
        out_v = rest[_NBUF]
        sems = rest[_NBUF + 1:]
        wid = lax.axis_index("s") * _NC + lax.axis_index("c")
        row0 = wid * _ROWS_PER_TILE
        pltpu.sync_copy(
            idx_hbm.at[pl.ds(wid * _GATHERS_PER_TILE, _GATHERS_PER_TILE)],
            idx_v)

        def src(g):
            return table_hbm.at[idx_v.at[g]]

        def fire(g, b):
            pltpu.async_copy(src(g), bufs[b], sems[b])

        def wait(g, b):
            pltpu.make_async_copy(src(g), bufs[b], sems[b]).wait()

        def reduce(buf, g):
            @pl.loop(0, _D // 32)
            def _reduce(gg):
                c0 = gg * 32
                for r in range(_ROWS_PER_GATHER):
                    base = r * _SEQ
                    ae = [None, None]
                    ao = [None, None]
                    for l in range(_SEQ):
                        v = plsc.bitcast(
                            buf[base + l, pl.ds(c0 // 2, _LANES)],
                            jnp.bfloat16)
                        e, o = plsc.unpack(
                            v, format=plsc.PackFormat.INTERLEAVED,
                            preferred_element_type=jnp.float32)
                        i = l % 2
                        ae[i] = e if ae[i] is None else ae[i] + e
                        ao[i] = o if ao[i] is None else ao[i] + o
                    row_out = g * _ROWS_PER_GATHER + r
                    out_v[row_out, pl.ds(c0, _LANES)] = (
                        (ae[0] + ae[1]) * (1.0 / _SEQ))
                    out_v[row_out, pl.ds(c0 + _LANES, _LANES)] = (
                        (ao[0] + ao[1]) * (1.0 / _SEQ))

        for b in range(_NBUF):
            fire(b, b)

        @pl.loop(0, _GATHERS_PER_TILE - _NBUF, step=_NBUF)
        def _main(g0):
            for b in range(_NBUF):
                g = g0 + b
                wait(g, b)
                reduce(bufs[b], g)
                fire(g + _NBUF, b)

        for b in range(_NBUF):
            g = _GATHERS_PER_TILE - _NBUF + b
            wait(g, b)
            reduce(bufs[b], g)

        pltpu.sync_copy(out_v, out_hbm.at[pl.ds(row0, _ROWS_PER_TILE)])

    return k(table_bf16, idx_flat)


def _tc_project(pooled, w, bias):
    blk = 512

    def body(p_ref, w_ref, b_ref, o_ref):
        o_ref[...] = lax.dot_general(
            p_ref[...], w_ref[...], (((1,), (0,)), ((), ())),
            preferred_element_type=jnp.float32,
            precision=lax.Precision.HIGHEST) + b_ref[...]

    return pl.pallas_call(
        body,
        grid=(_B // blk,),
        in_specs=[
            pl.BlockSpec((blk, _D), lambda i: (i, 0)),
            pl.BlockSpec((_D, _OUT), lambda i: (0, 0)),
            pl.BlockSpec((1, _OUT), lambda i: (0, 0)),
        ],
        out_specs=pl.BlockSpec((blk, _OUT), lambda i: (i, 0)),
        out_shape=jax.ShapeDtypeStruct((_B, _OUT), jnp.float32),
    )(pooled, w, bias.reshape(1, _OUT))


def kernel(x, table, W, b):
    xp = x.astype(jnp.int32).reshape(_B // _ROWS_PER_GATHER, _IDX_PER_GATHER)
    table_w = lax.bitcast_convert_type(
        table.astype(jnp.bfloat16).reshape(-1, _D // 2, 2), jnp.float32)
    pooled_perm = _sc_pool(table_w, xp)
    return _tc_project(pooled_perm, W[_PERM], b)


# speedup vs baseline: 13.5909x; 7.2292x over previous
import dataclasses
import functools

import jax
import jax.numpy as jnp
import numpy as np
from jax import lax
from jax.experimental import pallas as pl
from jax.experimental.pallas import tpu as pltpu
from jax.experimental.pallas import tpu_sc as plsc

_D = 128
_SEQ = 50
_B = 4096
_OUT = 512
_NC = 2
_NS = 16
_NW = _NC * _NS
_ROWS_PER_TILE = _B // _NW
_ROWS_PER_GATHER = 2
_IDX_PER_GATHER = _ROWS_PER_GATHER * _SEQ
_GATHERS_PER_TILE = _ROWS_PER_TILE // _ROWS_PER_GATHER
_LANES = 16
_NBUF = 4

_PERM = np.concatenate(
    [np.concatenate([g * 32 + np.arange(0, 32, 2),
                     g * 32 + np.arange(1, 32, 2)])
     for g in range(_D // 32)])


def _sc_pool(table_bf16, idx_flat):
    mesh = plsc.VectorSubcoreMesh(core_axis_name="c", subcore_axis_name="s")
    cp = pltpu.CompilerParams()
    if "needs_layout_passes" in pltpu.CompilerParams.__dataclass_fields__:
        cp = dataclasses.replace(cp, needs_layout_passes=False)
    if "use_tc_tiling_on_sc" in pltpu.CompilerParams.__dataclass_fields__:
        cp = dataclasses.replace(cp, use_tc_tiling_on_sc=False)

    @functools.partial(
        pl.kernel,
        mesh=mesh,
        compiler_params=cp,
        out_type=jax.ShapeDtypeStruct((_B, _D), jnp.float32),
        scratch_types=(
            [pltpu.VMEM((_GATHERS_PER_TILE, _IDX_PER_GATHER), jnp.int32)]
            + [pltpu.VMEM((_IDX_PER_GATHER, _D), jnp.float32)] * _NBUF
            + [pltpu.VMEM((_ROWS_PER_TILE, _D), jnp.float32)]
            + [pltpu.SemaphoreType.DMA] * _NBUF
        ),
    )
    def k(table_hbm, idx_hbm, out_hbm, idx_v, *rest):
        bufs = rest[:_NBUF]
        out_v = rest[_NBUF]
        sems = rest[_NBUF + 1:]
        wid = lax.axis_index("s") * _NC + lax.axis_index("c")
        row0 = wid * _ROWS_PER_TILE
        pltpu.sync_copy(
            idx_hbm.at[pl.ds(wid * _GATHERS_PER_TILE, _GATHERS_PER_TILE)],
            idx_v)

        def src(g):
            return table_hbm.at[idx_v.at[g]]

        def fire(g, b):
            pltpu.async_copy(src(g), bufs[b], sems[b])

        def wait(g, b):
            pltpu.make_async_copy(src(g), bufs[b], sems[b]).wait()

        def reduce(buf, g):
            @pl.loop(0, _D // _LANES)
            def _reduce(cc):
                c0 = cc * _LANES
                for r in range(_ROWS_PER_GATHER):
                    base = r * _SEQ
                    accs = [buf[base + a, pl.ds(c0, _LANES)]
                            for a in range(4)]
                    for l in range(4, _SEQ):
                        accs[l % 4] = accs[l % 4] + buf[base + l,
                                                        pl.ds(c0, _LANES)]
                    out_v[g * _ROWS_PER_GATHER + r, pl.ds(c0, _LANES)] = (
                        ((accs[0] + accs[1]) + (accs[2] + accs[3]))
                        * (1.0 / _SEQ))

        for b in range(_NBUF):
            fire(b, b)

        @pl.loop(0, _GATHERS_PER_TILE - _NBUF, step=_NBUF)
        def _main(g0):
            for b in range(_NBUF):
                g = g0 + b
                wait(g, b)
                reduce(bufs[b], g)
                fire(g + _NBUF, b)

        for b in range(_NBUF):
            g = _GATHERS_PER_TILE - _NBUF + b
            wait(g, b)
            reduce(bufs[b], g)

        pltpu.sync_copy(out_v, out_hbm.at[pl.ds(row0, _ROWS_PER_TILE)])

    return k(table_bf16, idx_flat)


def _tc_project(pooled, w, bias):
    blk = 512

    def body(p_ref, w_ref, b_ref, o_ref):
        o_ref[...] = lax.dot_general(
            p_ref[...], w_ref[...], (((1,), (0,)), ((), ())),
            preferred_element_type=jnp.float32,
            precision=lax.Precision.HIGHEST) + b_ref[...]

    return pl.pallas_call(
        body,
        grid=(_B // blk,),
        in_specs=[
            pl.BlockSpec((blk, _D), lambda i: (i, 0)),
            pl.BlockSpec((_D, _OUT), lambda i: (0, 0)),
            pl.BlockSpec((1, _OUT), lambda i: (0, 0)),
        ],
        out_specs=pl.BlockSpec((blk, _OUT), lambda i: (i, 0)),
        out_shape=jax.ShapeDtypeStruct((_B, _OUT), jnp.float32),
    )(pooled, w, bias.reshape(1, _OUT))


def kernel(x, table, W, b):
    xp = x.astype(jnp.int32).reshape(_B // _ROWS_PER_GATHER, _IDX_PER_GATHER)
    pooled = _sc_pool(table, xp)
    return _tc_project(pooled, W, b)
